# Initial kernel scaffold; baseline (speedup 1.0000x reference)
import jax
import jax.numpy as jnp
from jax.experimental import pallas as pl


def kernel(x, emb_tables, cbn_w, cbn_b, cbn_rm, cbn_rv, lin_W, lin_b, obn_w, obn_b, obn_rm, obn_rv):
    raise NotImplementedError("write your pallas kernel here")


# speedup vs baseline: 1.0997x; 1.0997x over previous
import jax
import jax.numpy as jnp
from jax.experimental import pallas as pl
from jax.experimental.pallas import tpu as pltpu

_B = 4096
_S = 50
_N_DISC = 26
_N_CONT = 13
_VOCAB = 1000
_EMB = 8
_BS = _B * _S
_D_OUT = 64
_EPS = 1e-5

_TB = 2048


def _dense_body(x_ref, g_ref, wc_ref, we_ref, b_ref, so_ref, to_ref, out_ref):
    xc = x_ref[:, _N_DISC:]
    z = jnp.dot(xc, wc_ref[...], preferred_element_type=jnp.float32)
    z = z + jnp.dot(g_ref[...], we_ref[...], preferred_element_type=jnp.float32)
    z = z + b_ref[...]
    out_ref[...] = jnp.maximum(z, 0.0) * so_ref[...] + to_ref[...]


def _dense_call(x2d, g, wc2, we2, b2, so, to):
    grid = (_BS // _TB,)
    return pl.pallas_call(
        _dense_body,
        grid=grid,
        in_specs=[
            pl.BlockSpec((_TB, _N_DISC + _N_CONT), lambda i: (i, 0)),
            pl.BlockSpec((_TB, _N_DISC * _EMB), lambda i: (i, 0)),
            pl.BlockSpec((_N_CONT, _D_OUT), lambda i: (0, 0)),
            pl.BlockSpec((_N_DISC * _EMB, _D_OUT), lambda i: (0, 0)),
            pl.BlockSpec((1, _D_OUT), lambda i: (0, 0)),
            pl.BlockSpec((1, _D_OUT), lambda i: (0, 0)),
            pl.BlockSpec((1, _D_OUT), lambda i: (0, 0)),
        ],
        out_specs=pl.BlockSpec((_TB, _D_OUT), lambda i: (i, 0)),
        out_shape=jax.ShapeDtypeStruct((_BS, _D_OUT), jnp.float32),
    )(x2d, g, wc2, we2, b2, so, to)


def kernel(x, emb_tables, cbn_w, cbn_b, cbn_rm, cbn_rv, lin_W, lin_b,
           obn_w, obn_b, obn_rm, obn_rv):
    x2d = x.reshape(_BS, _N_DISC + _N_CONT)

    s_c = cbn_w / jnp.sqrt(cbn_rv + _EPS)
    t_c = cbn_b - cbn_rm * s_c
    wc = lin_W[:, :_N_CONT]
    wc2 = (wc * s_c[None, :]).T
    b2 = lin_b + t_c @ wc.T
    we2 = lin_W[:, _N_CONT:].T

    s_o = obn_w / jnp.sqrt(obn_rv + _EPS)
    t_o = obn_b - obn_rm * s_o

    x_cat = x2d[:, :_N_DISC].astype(jnp.int32)
    g = emb_tables[jnp.arange(_N_DISC)[None, :], x_cat]
    g = g.reshape(_BS, _N_DISC * _EMB)

    out = _dense_call(x2d, g, wc2, we2,
                      b2.reshape(1, _D_OUT),
                      s_o.reshape(1, _D_OUT),
                      t_o.reshape(1, _D_OUT))
    return out.reshape(_B, _S, _D_OUT)


# speedup vs baseline: 74.3304x; 67.5914x over previous
import functools

import jax
import jax.numpy as jnp
from jax import lax
from jax.experimental import pallas as pl
from jax.experimental.pallas import tpu as pltpu
from jax.experimental.pallas import tpu_sc as plsc

_B = 4096
_S = 50
_N_DISC = 26
_N_CONT = 13
_VOCAB = 1000
_EMB = 8
_BS = _B * _S
_D_OUT = 64
_EPS = 1e-5

_NIDX = _BS * _N_DISC
_NW = 32
_PER_W = _NIDX // _NW
_CHUNK = 8320

_TB = 2048


def _sc_gather(table, idx):
    mesh = plsc.VectorSubcoreMesh(core_axis_name="c", subcore_axis_name="s")

    @functools.partial(
        pl.kernel,
        mesh=mesh,
        compiler_params=pltpu.CompilerParams(use_tc_tiling_on_sc=False),
        out_type=jax.ShapeDtypeStruct((_NIDX, _EMB), jnp.float32),
        scratch_types=[
            pltpu.VMEM((_CHUNK,), jnp.int32),
            pltpu.VMEM((_CHUNK, _EMB), jnp.float32),
            pltpu.SemaphoreType.DMA,
        ],
    )
    def gather_kernel(table_hbm, idx_hbm, out_hbm, idx_v, rows_v, sem):
        wid = lax.axis_index("s") * 2 + lax.axis_index("c")
        base = wid * _PER_W

        def body(i, carry):
            off = base + i * _CHUNK
            pltpu.sync_copy(idx_hbm.at[pl.ds(off, _CHUNK)], idx_v)
            pltpu.async_copy(table_hbm.at[idx_v], rows_v, sem).wait()
            pltpu.sync_copy(rows_v, out_hbm.at[pl.ds(off, _CHUNK)])
            return carry

        lax.fori_loop(0, _PER_W // _CHUNK, body, 0)

    return gather_kernel(table, idx)


def _dense_body(x_ref, g_ref, wc_ref, we_ref, b_ref, so_ref, to_ref, out_ref):
    xc = x_ref[:, _N_DISC:]
    z = jnp.dot(xc, wc_ref[...], preferred_element_type=jnp.float32)
    z = z + jnp.dot(g_ref[...], we_ref[...], preferred_element_type=jnp.float32)
    z = z + b_ref[...]
    out_ref[...] = jnp.maximum(z, 0.0) * so_ref[...] + to_ref[...]


def _dense_call(x2d, g, wc2, we2, b2, so, to):
    return pl.pallas_call(
        _dense_body,
        grid=(_BS // _TB,),
        in_specs=[
            pl.BlockSpec((_TB, _N_DISC + _N_CONT), lambda i: (i, 0)),
            pl.BlockSpec((_TB, _N_DISC * _EMB), lambda i: (i, 0)),
            pl.BlockSpec((_N_CONT, _D_OUT), lambda i: (0, 0)),
            pl.BlockSpec((_N_DISC * _EMB, _D_OUT), lambda i: (0, 0)),
            pl.BlockSpec((1, _D_OUT), lambda i: (0, 0)),
            pl.BlockSpec((1, _D_OUT), lambda i: (0, 0)),
            pl.BlockSpec((1, _D_OUT), lambda i: (0, 0)),
        ],
        out_specs=pl.BlockSpec((_TB, _D_OUT), lambda i: (i, 0)),
        out_shape=jax.ShapeDtypeStruct((_BS, _D_OUT), jnp.float32),
    )(x2d, g, wc2, we2, b2, so, to)


def kernel(x, emb_tables, cbn_w, cbn_b, cbn_rm, cbn_rv, lin_W, lin_b,
           obn_w, obn_b, obn_rm, obn_rv):
    x2d = x.reshape(_BS, _N_DISC + _N_CONT)

    s_c = cbn_w / jnp.sqrt(cbn_rv + _EPS)
    t_c = cbn_b - cbn_rm * s_c
    wc = lin_W[:, :_N_CONT]
    wc2 = (wc * s_c[None, :]).T
    b2 = lin_b + t_c @ wc.T
    we2 = lin_W[:, _N_CONT:].T

    s_o = obn_w / jnp.sqrt(obn_rv + _EPS)
    t_o = obn_b - obn_rm * s_o

    idx = (x2d[:, :_N_DISC].astype(jnp.int32)
           + (jnp.arange(_N_DISC, dtype=jnp.int32) * _VOCAB)[None, :])
    idx = idx.reshape(_NIDX)
    table = emb_tables.reshape(_N_DISC * _VOCAB, _EMB)

    g = _sc_gather(table, idx).reshape(_BS, _N_DISC * _EMB)

    out = _dense_call(x2d, g, wc2, we2,
                      b2.reshape(1, _D_OUT),
                      s_o.reshape(1, _D_OUT),
                      t_o.reshape(1, _D_OUT))
    return out.reshape(_B, _S, _D_OUT)
